# Initial kernel scaffold; baseline (speedup 1.0000x reference)
import jax
import jax.numpy as jnp
from jax.experimental import pallas as pl


def kernel(query_coords, sensor_coords, sensor_features, sigma):
    raise NotImplementedError("write your pallas kernel here")


# speedup vs baseline: 14.5140x; 14.5140x over previous
import functools

import jax
import jax.numpy as jnp
from jax.experimental import pallas as pl
from jax.experimental.pallas import tpu as pltpu

_K = 8
_TQ = 512


def _round_bf16(x):
    u = jax.lax.bitcast_convert_type(x, jnp.uint32)
    lsb = jax.lax.shift_right_logical(u, jnp.uint32(16)) & jnp.uint32(1)
    r = (u + jnp.uint32(0x7FFF) + lsb) & jnp.uint32(0xFFFF0000)
    return jax.lax.bitcast_convert_type(r, jnp.float32)


def _tc_body(coef_ref, q_ref, s_ref, f_ref, o_ref):
    q = q_ref[...]
    s = s_ref[...]
    qb = _round_bf16(q)
    sb = _round_bf16(s)
    cross = (qb[:, 0:1] * sb[0:1, :] + qb[:, 1:2] * sb[1:2, :]
             + qb[:, 2:3] * sb[2:3, :])
    q2 = jnp.sum(q * q, axis=1, keepdims=True)
    s2 = jnp.sum(s * s, axis=0, keepdims=True)
    d2 = jnp.maximum((q2 + s2) - 2.0 * cross, 1e-12)

    ns = d2.shape[1]
    lane = jax.lax.broadcasted_iota(jnp.int32, d2.shape, 1)
    work = d2
    sel = jnp.zeros(d2.shape, dtype=jnp.bool_)
    for _ in range(_K):
        m = jnp.min(work, axis=1, keepdims=True)
        cand = jnp.where(work == m, lane, ns)
        i = jnp.min(cand, axis=1, keepdims=True)
        hit = lane == i
        sel = sel | hit
        work = jnp.where(hit, jnp.float32(jnp.inf), work)

    coef = coef_ref[0, 0]
    w = jnp.where(sel, jnp.exp(coef * d2), 0.0)
    wsum = jnp.sum(w, axis=1, keepdims=True) + 1e-5
    wn = w / wsum
    feats = f_ref[0]
    o_ref[...] = jax.lax.dot_general(
        wn, feats, (((1,), (0,)), ((), ())),
        precision=jax.lax.Precision.HIGHEST,
        preferred_element_type=jnp.float32)


@jax.jit
def kernel(query_coords, sensor_coords, sensor_features, sigma):
    B, Nq, _ = query_coords.shape
    Ns, F = sensor_features.shape[1], sensor_features.shape[2]
    nq_blocks = Nq // _TQ

    q_flat = query_coords.reshape(B * Nq, 3)
    s_t = sensor_coords.T
    sigma_safe = jax.nn.softplus(sigma) + 0.01
    coef = (-1.0 / (2.0 * sigma_safe * sigma_safe)).reshape(1, 1)

    grid = (B, nq_blocks)
    out = pl.pallas_call(
        _tc_body,
        grid_spec=pltpu.PrefetchScalarGridSpec(
            num_scalar_prefetch=0,
            grid=grid,
            in_specs=[
                pl.BlockSpec(memory_space=pltpu.SMEM),
                pl.BlockSpec((_TQ, 3), lambda b, i: (b * nq_blocks + i, 0)),
                pl.BlockSpec((3, Ns), lambda b, i: (0, 0)),
                pl.BlockSpec((1, Ns, F), lambda b, i: (b, 0, 0)),
            ],
            out_specs=pl.BlockSpec((_TQ, F), lambda b, i: (b * nq_blocks + i, 0)),
        ),
        out_shape=jax.ShapeDtypeStruct((B * Nq, F), jnp.float32),
    )(coef, q_flat, s_t, sensor_features)
    return out.reshape(B, Nq, F)


# speedup vs baseline: 17.0229x; 1.1729x over previous
import functools

import jax
import jax.numpy as jnp
from jax import lax
from jax.experimental import pallas as pl
from jax.experimental.pallas import tpu as pltpu
from jax.experimental.pallas import tpu_sc as plsc

_K = 8
_TQ = 512
_G = 16


def _round_bf16(x):
    u = jax.lax.bitcast_convert_type(x, jnp.uint32)
    lsb = jax.lax.shift_right_logical(u, jnp.uint32(16)) & jnp.uint32(1)
    r = (u + jnp.uint32(0x7FFF) + lsb) & jnp.uint32(0xFFFF0000)
    return jax.lax.bitcast_convert_type(r, jnp.float32)


def _tc_body(coef_ref, q_ref, s_ref, oi_ref, ow_ref):
    q = q_ref[...]
    s = s_ref[...]
    qb = _round_bf16(q)
    sb = _round_bf16(s)
    cross = (qb[:, 0:1] * sb[0:1, :] + qb[:, 1:2] * sb[1:2, :]
             + qb[:, 2:3] * sb[2:3, :])
    q2 = jnp.sum(q * q, axis=1, keepdims=True)
    s2 = jnp.sum(s * s, axis=0, keepdims=True)
    d2 = jnp.maximum((q2 + s2) - 2.0 * cross, 1e-12)

    ns = d2.shape[1]
    lane = jax.lax.broadcasted_iota(jnp.int32, d2.shape, 1)
    work = d2
    vals = []
    idxs = []
    for _ in range(_K):
        m = jnp.min(work, axis=1, keepdims=True)
        cand = jnp.where(work == m, lane, ns)
        i = jnp.min(cand, axis=1, keepdims=True)
        vals.append(m)
        idxs.append(i)
        work = jnp.where(lane == i, jnp.float32(jnp.inf), work)

    v8 = jnp.concatenate(vals, axis=1)
    i8 = jnp.concatenate(idxs, axis=1)
    coef = coef_ref[0, 0]
    w = jnp.exp(coef * v8)
    wn = w / (jnp.sum(w, axis=1, keepdims=True) + 1e-5)
    b = pl.program_id(0)
    oi_ref[...] = i8 + b * ns
    ow_ref[...] = wn


def _sc_body(f_ref, i_ref, w_ref, o_ref, idx_v, w_v, rows_v, out_v, sem):
    nc = 2
    wid = lax.axis_index("s") * nc + lax.axis_index("c")
    qpw = 512
    nchunk = qpw // _G

    def chunk(c, _):
        qbase = wid * qpw + c * _G
        pltpu.sync_copy(i_ref.at[pl.ds(pl.multiple_of(qbase * _K, 128), _G * _K)],
                        idx_v)
        pltpu.sync_copy(w_ref.at[pl.ds(pl.multiple_of(qbase * _K, 128), _G * _K)],
                        w_v)
        pltpu.async_copy(f_ref.at[idx_v], rows_v, sem).wait()
        for qp in range(_G // 2):
            wvec = w_v[pl.ds(qp * 16, 16)]
            for h in range(2):
                qq = qp * 2 + h
                accs = []
                for j in range(4):
                    acc = jnp.zeros((16,), jnp.float32)
                    accs.append(acc)
                for k in range(_K):
                    wb = wvec.at[jnp.full((16,), h * 8 + k, jnp.int32)].get(
                        mode='promise_in_bounds')
                    for j in range(4):
                        row = rows_v[qq * _K + k, pl.ds(j * 16, 16)]
                        accs[j] = accs[j] + wb * row
                for j in range(4):
                    out_v[qq, pl.ds(j * 16, 16)] = accs[j]
        pltpu.sync_copy(out_v, o_ref.at[pl.ds(pl.multiple_of(qbase, _G), _G)])
        return ()

    lax.fori_loop(0, nchunk, chunk, (), unroll=False)


@jax.jit
def kernel(query_coords, sensor_coords, sensor_features, sigma):
    B, Nq, _ = query_coords.shape
    Ns, F = sensor_features.shape[1], sensor_features.shape[2]
    nq_blocks = Nq // _TQ

    q_flat = query_coords.reshape(B * Nq, 3)
    s_t = sensor_coords.T
    sigma_safe = jax.nn.softplus(sigma) + 0.01
    coef = (-1.0 / (2.0 * sigma_safe * sigma_safe)).reshape(1, 1)

    idx8, w8 = pl.pallas_call(
        _tc_body,
        grid_spec=pltpu.PrefetchScalarGridSpec(
            num_scalar_prefetch=0,
            grid=(B, nq_blocks),
            in_specs=[
                pl.BlockSpec(memory_space=pltpu.SMEM),
                pl.BlockSpec((_TQ, 3), lambda b, i: (b * nq_blocks + i, 0)),
                pl.BlockSpec((3, Ns), lambda b, i: (0, 0)),
            ],
            out_specs=[
                pl.BlockSpec((_TQ, _K), lambda b, i: (b * nq_blocks + i, 0)),
                pl.BlockSpec((_TQ, _K), lambda b, i: (b * nq_blocks + i, 0)),
            ],
        ),
        out_shape=[
            jax.ShapeDtypeStruct((B * Nq, _K), jnp.int32),
            jax.ShapeDtypeStruct((B * Nq, _K), jnp.float32),
        ],
    )(coef, q_flat, s_t)

    feats_flat = jnp.pad(sensor_features.reshape(B * Ns, F),
                         ((0, 0), (0, 128 - F)))
    idx_flat = idx8.reshape(B * Nq * _K)
    w_flat = w8.reshape(B * Nq * _K)

    sc = functools.partial(
        pl.kernel,
        out_type=jax.ShapeDtypeStruct((B * Nq, F), jnp.float32),
        mesh=plsc.VectorSubcoreMesh(core_axis_name="c", subcore_axis_name="s"),
        scratch_types=[
            pltpu.VMEM((_G * _K,), jnp.int32),
            pltpu.VMEM((_G * _K,), jnp.float32),
            pltpu.VMEM((_G * _K, 128), jnp.float32),
            pltpu.VMEM((_G, F), jnp.float32),
            pltpu.SemaphoreType.DMA,
        ],
    )(_sc_body)

    out = sc(feats_flat, idx_flat, w_flat)
    return out.reshape(B, Nq, F)


# speedup vs baseline: 20.3808x; 1.1973x over previous
import functools

import jax
import jax.numpy as jnp
from jax import lax
from jax.experimental import pallas as pl
from jax.experimental.pallas import tpu as pltpu
from jax.experimental.pallas import tpu_sc as plsc

_K = 8
_TQ = 512
_G = 16


def _round_bf16(x):
    u = jax.lax.bitcast_convert_type(x, jnp.uint32)
    lsb = jax.lax.shift_right_logical(u, jnp.uint32(16)) & jnp.uint32(1)
    r = (u + jnp.uint32(0x7FFF) + lsb) & jnp.uint32(0xFFFF0000)
    return jax.lax.bitcast_convert_type(r, jnp.float32)


def _tc_body(coef_ref, q_ref, s_ref, oi_ref, ow_ref):
    q = q_ref[...]
    s = s_ref[...]
    qb = _round_bf16(q)
    sb = _round_bf16(s)
    cross2 = jnp.dot(-2.0 * qb, sb,
                     preferred_element_type=jnp.float32)
    q2 = jnp.sum(q * q, axis=1, keepdims=True)
    s2 = jnp.sum(s * s, axis=0, keepdims=True)
    d2 = jnp.maximum((q2 + s2) + cross2, 1e-12)

    ns = d2.shape[1]
    lane_f = jax.lax.broadcasted_iota(jnp.int32, d2.shape, 1).astype(jnp.float32)
    big = jnp.float32(float(ns))
    work = d2
    vals = []
    idxs = []
    for _ in range(_K):
        m = jnp.min(work, axis=1, keepdims=True)
        cand = jnp.where(work == m, lane_f, big)
        i = jnp.min(cand, axis=1, keepdims=True)
        vals.append(m)
        idxs.append(i)
        work = jnp.where(cand == i, jnp.float32(jnp.inf), work)

    v8 = jnp.concatenate(vals, axis=1)
    i8 = jnp.concatenate(idxs, axis=1).astype(jnp.int32)
    coef = coef_ref[0, 0]
    w = jnp.exp(coef * v8)
    wn = w / (jnp.sum(w, axis=1, keepdims=True) + 1e-5)
    b = pl.program_id(0)
    oi_ref[...] = i8 + b * ns
    ow_ref[...] = wn


def _sc_body(f_ref, i_ref, w_ref, o_ref, idx_v, w_v, rows_v, out_v, sem):
    nc = 2
    wid = lax.axis_index("s") * nc + lax.axis_index("c")
    qpw = 512
    nchunk = qpw // _G

    def chunk(c, _):
        qbase = wid * qpw + c * _G
        pltpu.sync_copy(i_ref.at[pl.ds(pl.multiple_of(qbase * _K, 128), _G * _K)],
                        idx_v)
        pltpu.sync_copy(w_ref.at[pl.ds(pl.multiple_of(qbase * _K, 128), _G * _K)],
                        w_v)
        pltpu.async_copy(f_ref.at[idx_v], rows_v, sem).wait()
        for qp in range(_G // 2):
            wvec = w_v[pl.ds(qp * 16, 16)]
            for h in range(2):
                qq = qp * 2 + h
                accs = []
                for j in range(4):
                    acc = jnp.zeros((16,), jnp.float32)
                    accs.append(acc)
                for k in range(_K):
                    wb = wvec.at[jnp.full((16,), h * 8 + k, jnp.int32)].get(
                        mode='promise_in_bounds')
                    for j in range(4):
                        row = rows_v[qq * _K + k, pl.ds(j * 16, 16)]
                        accs[j] = accs[j] + wb * row
                for j in range(4):
                    out_v[qq, pl.ds(j * 16, 16)] = accs[j]
        pltpu.sync_copy(out_v, o_ref.at[pl.ds(pl.multiple_of(qbase, _G), _G)])
        return ()

    lax.fori_loop(0, nchunk, chunk, (), unroll=False)


@jax.jit
def kernel(query_coords, sensor_coords, sensor_features, sigma):
    B, Nq, _ = query_coords.shape
    Ns, F = sensor_features.shape[1], sensor_features.shape[2]
    nq_blocks = Nq // _TQ

    q_flat = query_coords.reshape(B * Nq, 3)
    s_t = sensor_coords.T
    sigma_safe = jax.nn.softplus(sigma) + 0.01
    coef = (-1.0 / (2.0 * sigma_safe * sigma_safe)).reshape(1, 1)

    idx8, w8 = pl.pallas_call(
        _tc_body,
        grid_spec=pltpu.PrefetchScalarGridSpec(
            num_scalar_prefetch=0,
            grid=(B, nq_blocks),
            in_specs=[
                pl.BlockSpec(memory_space=pltpu.SMEM),
                pl.BlockSpec((_TQ, 3), lambda b, i: (b * nq_blocks + i, 0)),
                pl.BlockSpec((3, Ns), lambda b, i: (0, 0)),
            ],
            out_specs=[
                pl.BlockSpec((_TQ, _K), lambda b, i: (b * nq_blocks + i, 0)),
                pl.BlockSpec((_TQ, _K), lambda b, i: (b * nq_blocks + i, 0)),
            ],
        ),
        out_shape=[
            jax.ShapeDtypeStruct((B * Nq, _K), jnp.int32),
            jax.ShapeDtypeStruct((B * Nq, _K), jnp.float32),
        ],
    )(coef, q_flat, s_t)

    feats_flat = jnp.pad(sensor_features.reshape(B * Ns, F),
                         ((0, 0), (0, 128 - F)))
    idx_flat = idx8.reshape(B * Nq * _K)
    w_flat = w8.reshape(B * Nq * _K)

    sc = functools.partial(
        pl.kernel,
        out_type=jax.ShapeDtypeStruct((B * Nq, F), jnp.float32),
        mesh=plsc.VectorSubcoreMesh(core_axis_name="c", subcore_axis_name="s"),
        scratch_types=[
            pltpu.VMEM((_G * _K,), jnp.int32),
            pltpu.VMEM((_G * _K,), jnp.float32),
            pltpu.VMEM((_G * _K, 128), jnp.float32),
            pltpu.VMEM((_G, F), jnp.float32),
            pltpu.SemaphoreType.DMA,
        ],
    )(_sc_body)

    out = sc(feats_flat, idx_flat, w_flat)
    return out.reshape(B, Nq, F)


# speedup vs baseline: 22.8015x; 1.1188x over previous
import functools

import jax
import jax.numpy as jnp
from jax import lax
from jax.experimental import pallas as pl
from jax.experimental.pallas import tpu as pltpu
from jax.experimental.pallas import tpu_sc as plsc

_K = 8
_TQ = 512
_G = 16


def _round_bf16(x):
    u = jax.lax.bitcast_convert_type(x, jnp.uint32)
    lsb = jax.lax.shift_right_logical(u, jnp.uint32(16)) & jnp.uint32(1)
    r = (u + jnp.uint32(0x7FFF) + lsb) & jnp.uint32(0xFFFF0000)
    return jax.lax.bitcast_convert_type(r, jnp.float32)


def _tc_body(coef_ref, q_ref, s_ref, oi_ref, ow_ref):
    q = q_ref[...]
    s = s_ref[...]
    qb = _round_bf16(q)
    sb = _round_bf16(s)
    cross2 = jnp.dot(-2.0 * qb, sb,
                     preferred_element_type=jnp.float32)
    q2 = jnp.sum(q * q, axis=1, keepdims=True)
    s2 = jnp.sum(s * s, axis=0, keepdims=True)
    d2 = jnp.maximum((q2 + s2) + cross2, 1e-12)

    ns = d2.shape[1]
    lane_f = jax.lax.broadcasted_iota(jnp.int32, d2.shape, 1).astype(jnp.float32)
    big = jnp.float32(float(ns))
    work = d2
    vals = []
    idxs = []
    for _ in range(_K):
        m = jnp.min(work, axis=1, keepdims=True)
        cand = jnp.where(work == m, lane_f, big)
        i = jnp.min(cand, axis=1, keepdims=True)
        vals.append(m)
        idxs.append(i)
        work = jnp.where(cand == i, jnp.float32(jnp.inf), work)

    v8 = jnp.concatenate(vals, axis=1)
    i8 = jnp.concatenate(idxs, axis=1).astype(jnp.int32)
    coef = coef_ref[0, 0]
    w = jnp.exp(coef * v8)
    wn = w / (jnp.sum(w, axis=1, keepdims=True) + 1e-5)
    b = pl.program_id(0)
    oi_ref[...] = i8 + b * ns
    ow_ref[...] = wn


def _sc_body(f_ref, i_ref, w_ref, o_ref,
             idx0_v, idx1_v, w0_v, w1_v, rows0_v, rows1_v, out_v,
             sem0, sem1):
    nc = 2
    wid = lax.axis_index("s") * nc + lax.axis_index("c")
    qpw = 512
    nchunk = qpw // _G
    slots = ((idx0_v, w0_v, rows0_v, sem0), (idx1_v, w1_v, rows1_v, sem1))

    def prefetch(c, slot):
        idx_v, w_v, rows_v, sem = slot
        qbase = wid * qpw + c * _G
        pltpu.sync_copy(i_ref.at[pl.ds(pl.multiple_of(qbase * _K, 128), _G * _K)],
                        idx_v)
        pltpu.sync_copy(w_ref.at[pl.ds(pl.multiple_of(qbase * _K, 128), _G * _K)],
                        w_v)
        pltpu.async_copy(f_ref.at[idx_v], rows_v, sem)

    def compute(c, slot):
        idx_v, w_v, rows_v, sem = slot
        qbase = wid * qpw + c * _G
        pltpu.make_async_copy(f_ref.at[idx_v], rows_v, sem).wait()
        for qp in range(_G // 2):
            wvec = w_v[pl.ds(qp * 16, 16)]
            for h in range(2):
                qq = qp * 2 + h
                accs = [jnp.zeros((16,), jnp.float32) for _ in range(4)]
                for k in range(_K):
                    wb = wvec.at[jnp.full((16,), h * 8 + k, jnp.int32)].get(
                        mode='promise_in_bounds')
                    for j in range(4):
                        row = rows_v[qq * _K + k, pl.ds(j * 16, 16)]
                        accs[j] = accs[j] + wb * row
                for j in range(4):
                    out_v[qq, pl.ds(j * 16, 16)] = accs[j]
        pltpu.sync_copy(out_v, o_ref.at[pl.ds(pl.multiple_of(qbase, _G), _G)])

    prefetch(0, slots[0])

    def pair(p, _):
        c0 = p * 2
        prefetch(c0 + 1, slots[1])
        compute(c0, slots[0])

        @pl.when(c0 + 2 < nchunk)
        def _():
            prefetch(c0 + 2, slots[0])

        compute(c0 + 1, slots[1])
        return ()

    lax.fori_loop(0, nchunk // 2, pair, ())


@jax.jit
def kernel(query_coords, sensor_coords, sensor_features, sigma):
    B, Nq, _ = query_coords.shape
    Ns, F = sensor_features.shape[1], sensor_features.shape[2]
    nq_blocks = Nq // _TQ

    q_flat = query_coords.reshape(B * Nq, 3)
    s_t = sensor_coords.T
    sigma_safe = jax.nn.softplus(sigma) + 0.01
    coef = (-1.0 / (2.0 * sigma_safe * sigma_safe)).reshape(1, 1)

    idx8, w8 = pl.pallas_call(
        _tc_body,
        grid_spec=pltpu.PrefetchScalarGridSpec(
            num_scalar_prefetch=0,
            grid=(B, nq_blocks),
            in_specs=[
                pl.BlockSpec(memory_space=pltpu.SMEM),
                pl.BlockSpec((_TQ, 3), lambda b, i: (b * nq_blocks + i, 0)),
                pl.BlockSpec((3, Ns), lambda b, i: (0, 0)),
            ],
            out_specs=[
                pl.BlockSpec((_TQ, _K), lambda b, i: (b * nq_blocks + i, 0)),
                pl.BlockSpec((_TQ, _K), lambda b, i: (b * nq_blocks + i, 0)),
            ],
        ),
        out_shape=[
            jax.ShapeDtypeStruct((B * Nq, _K), jnp.int32),
            jax.ShapeDtypeStruct((B * Nq, _K), jnp.float32),
        ],
    )(coef, q_flat, s_t)

    feats_flat = jnp.pad(sensor_features.reshape(B * Ns, F),
                         ((0, 0), (0, 128 - F)))
    idx_flat = idx8.reshape(B * Nq * _K)
    w_flat = w8.reshape(B * Nq * _K)

    sc = functools.partial(
        pl.kernel,
        out_type=jax.ShapeDtypeStruct((B * Nq, F), jnp.float32),
        mesh=plsc.VectorSubcoreMesh(core_axis_name="c", subcore_axis_name="s"),
        scratch_types=[
            pltpu.VMEM((_G * _K,), jnp.int32),
            pltpu.VMEM((_G * _K,), jnp.int32),
            pltpu.VMEM((_G * _K,), jnp.float32),
            pltpu.VMEM((_G * _K,), jnp.float32),
            pltpu.VMEM((_G * _K, 128), jnp.float32),
            pltpu.VMEM((_G * _K, 128), jnp.float32),
            pltpu.VMEM((_G, F), jnp.float32),
            pltpu.SemaphoreType.DMA,
            pltpu.SemaphoreType.DMA,
        ],
    )(_sc_body)

    out = sc(feats_flat, idx_flat, w_flat)
    return out.reshape(B, Nq, F)
